# baseline (speedup 1.0000x reference)
import functools

import jax
import jax.numpy as jnp
from jax import lax
from jax.experimental import pallas as pl
from jax.experimental.pallas import tpu as pltpu
from jax.experimental.pallas import tpu_sc as plsc

LANES = 16
NFEAT = 340
H3, H2, H1 = 0, 64, 80
O1, O2, O3, O4 = 0, 4, 20, 84


def _encoder_body(L, groups, seq_hbm, out_hbm, seq_v, hist4_v, hist123_v,
                  stage_v, sem_in0, sem_in1, sem_out0, sem_out1):
    seq_blk = LANES * L
    stage_blk = LANES * NFEAT
    n_chunks = L // LANES
    iota = lax.iota(jnp.int32, LANES)
    ones_f = jnp.full((LANES,), 1.0, jnp.float32)
    zeros_f = jnp.zeros((LANES,), jnp.float32)
    quad = iota >> 2
    gather_dn = lax.GatherDimensionNumbers(
        offset_dims=(), collapsed_slice_dims=(0,), start_index_map=(0,))

    def vrot(x, idx2d):
        return lax.gather(x, idx2d, gather_dn, slice_sizes=(1,),
                          mode=lax.GatherScatterMode.PROMISE_IN_BOUNDS)

    wid = lax.axis_index("s") * 2 + lax.axis_index("c")
    row0 = wid * (groups * LANES)

    sems_in = (sem_in0, sem_in1)
    sems_out = (sem_out0, sem_out1)

    def start_in(g, b):
        return pltpu.async_copy(
            seq_hbm.at[pl.ds((row0 + g * LANES) * L, seq_blk)],
            seq_v.at[pl.ds(b * seq_blk, seq_blk)],
            sems_in[b])

    in_copies = [start_in(0, 0), None]
    out_copies = [None, None]

    for i in range(16):
        hist4_v[pl.ds(i * LANES, LANES)] = zeros_f
    for i in range(6):
        hist123_v[pl.ds(i * LANES, LANES)] = zeros_f

    inv4 = jnp.full((LANES,), 1.0 / (L - 3), jnp.float32)
    inv3 = jnp.full((LANES,), 1.0 / (L - 2), jnp.float32)
    inv2 = jnp.full((LANES,), 1.0 / (L - 1), jnp.float32)
    inv1 = jnp.full((LANES,), 1.0 / L, jnp.float32)

    for g in range(groups):
        b = g & 1
        in_copies[b].wait()
        if g + 1 < groups:
            in_copies[1 - b] = start_in(g + 1, 1 - b)
        if out_copies[b] is not None:
            out_copies[b].wait()

        def row_body(r, _):
            rowbase = b * seq_blk + r * L
            rstage = b * stage_blk + r * NFEAT

            def windows(p):
                a = seq_v[pl.ds(p, LANES)]
                bb = seq_v[pl.ds(p + 1, LANES)]
                cc = seq_v[pl.ds(p + 2, LANES)]
                dd = seq_v[pl.ds(p + 3, LANES)]
                return a, bb, cc, dd

            rot1 = (((iota + 1) & (LANES - 1)))[:, None]
            first_b = seq_v[pl.ds(rowbase + 1, LANES)]

            @plsc.parallel_loop(0, n_chunks - 1, unroll=4, carry=first_b)
            def chunk_body(ch, bcur):
                p = rowbase + ch * LANES
                a = seq_v[pl.ds(p, LANES)]
                dd = seq_v[pl.ds(p + 3, LANES)]
                bnext = seq_v[pl.ds(p + LANES + 1, LANES)]
                cc = jnp.where(iota < LANES - 1, vrot(bcur, rot1),
                               vrot(bnext, rot1))
                code = (((a << 2) + bcur) << 2) + cc
                code = (code << 2) + dd
                plsc.addupdate_scatter(hist4_v, [code], ones_f)
                return bnext

            a, bb, cc, dd = windows(rowbase + (n_chunks - 1) * LANES)
            t1 = (a << 2) + bb
            code = (((t1 << 2) + cc) << 2) + dd
            plsc.addupdate_scatter(hist4_v, [code], ones_f,
                                   mask=iota < LANES - 3)
            v3 = (t1 << 2) + cc + H3
            v2 = t1 + H2
            v1 = a + H1
            corr = jnp.where(iota == LANES - 3, v3,
                             jnp.where(iota == LANES - 2, v2, v1))
            plsc.addupdate_scatter(hist123_v, [corr], ones_f,
                                   mask=iota >= LANES - 3)

            for i in range(16):
                v = hist4_v[pl.ds(i * LANES, LANES)]
                plsc.addupdate_scatter(hist123_v, [quad + (H3 + 4 * i)], v)
                stage_v[pl.ds(rstage + O4 + i * LANES, LANES)] = v * inv4
            for i in range(16):
                hist4_v[pl.ds(i * LANES, LANES)] = zeros_f
            for i in range(4):
                v = hist123_v[pl.ds(H3 + i * LANES, LANES)]
                plsc.addupdate_scatter(hist123_v, [quad + (H2 + 4 * i)], v)
                stage_v[pl.ds(rstage + O3 + i * LANES, LANES)] = v * inv3
            for i in range(4):
                hist123_v[pl.ds(H3 + i * LANES, LANES)] = zeros_f
            v = hist123_v[pl.ds(H2, LANES)]
            plsc.addupdate_scatter(hist123_v, [quad + H1], v)
            stage_v[pl.ds(rstage + O2, LANES)] = v * inv2
            hist123_v[pl.ds(H2, LANES)] = zeros_f
            v = hist123_v[pl.ds(H1 - 12, LANES)]
            plsc.store_scatter(stage_v, [(iota - 12) + rstage], v * inv1,
                               mask=iota >= 12)
            hist123_v[pl.ds(H1 - 12, LANES)] = zeros_f
            return 0

        lax.fori_loop(0, LANES, row_body, 0)

        out_copies[b] = pltpu.async_copy(
            stage_v.at[pl.ds(b * stage_blk, stage_blk)],
            out_hbm.at[pl.ds((row0 + g * LANES) * NFEAT, stage_blk)],
            sems_out[b])

    for cp in out_copies:
        if cp is not None:
            cp.wait()


def kernel(sequences):
    B, L = sequences.shape
    groups = B // (32 * LANES)
    mesh = plsc.VectorSubcoreMesh(core_axis_name="c", subcore_axis_name="s")
    run = pl.kernel(
        functools.partial(_encoder_body, L, groups),
        out_type=jax.ShapeDtypeStruct((B * NFEAT,), jnp.float32),
        mesh=mesh,
        scratch_types=[
            pltpu.VMEM((2 * LANES * L,), jnp.int32),
            pltpu.VMEM((256,), jnp.float32),
            pltpu.VMEM((96,), jnp.float32),
            pltpu.VMEM((2 * LANES * NFEAT,), jnp.float32),
            pltpu.SemaphoreType.DMA,
            pltpu.SemaphoreType.DMA,
            pltpu.SemaphoreType.DMA,
            pltpu.SemaphoreType.DMA,
        ],
        compiler_params=pltpu.CompilerParams(
            needs_layout_passes=False, disable_bounds_checks=True),
    )
    out = run(sequences.reshape(B * L))
    return out.reshape(B, NFEAT)


# speedup vs baseline: 1.0119x; 1.0119x over previous
import functools

import jax
import jax.numpy as jnp
from jax import lax
from jax.experimental import pallas as pl
from jax.experimental.pallas import tpu as pltpu
from jax.experimental.pallas import tpu_sc as plsc

LANES = 16
NFEAT = 340
H3, H2, H1 = 0, 64, 80
O1, O2, O3, O4 = 0, 4, 20, 84


def _encoder_body(L, groups, seq_hbm, out_hbm, seq_v, hist4_v, hist123_v,
                  stage_v, sem_in0, sem_in1, sem_out0, sem_out1):
    seq_blk = LANES * L
    stage_blk = LANES * NFEAT
    n_chunks = L // LANES
    iota = lax.iota(jnp.int32, LANES)
    ones_f = jnp.full((LANES,), 1.0, jnp.float32)
    zeros_f = jnp.zeros((LANES,), jnp.float32)
    quad = iota >> 2
    gather_dn = lax.GatherDimensionNumbers(
        offset_dims=(), collapsed_slice_dims=(0,), start_index_map=(0,))

    def vrot(x, idx2d):
        return lax.gather(x, idx2d, gather_dn, slice_sizes=(1,),
                          mode=lax.GatherScatterMode.PROMISE_IN_BOUNDS)

    wid = lax.axis_index("s") * 2 + lax.axis_index("c")
    row0 = wid * (groups * LANES)

    sems_in = (sem_in0, sem_in1)
    sems_out = (sem_out0, sem_out1)

    def start_in(g, b):
        return pltpu.async_copy(
            seq_hbm.at[pl.ds((row0 + g * LANES) * L, seq_blk)],
            seq_v.at[pl.ds(b * seq_blk, seq_blk)],
            sems_in[b])

    in_copies = [start_in(0, 0), None]
    out_copies = [None, None]

    for i in range(16):
        hist4_v[pl.ds(i * LANES, LANES)] = zeros_f
    for i in range(6):
        hist123_v[pl.ds(i * LANES, LANES)] = zeros_f

    inv4 = jnp.full((LANES,), 1.0 / (L - 3), jnp.float32)
    inv3 = jnp.full((LANES,), 1.0 / (L - 2), jnp.float32)
    inv2 = jnp.full((LANES,), 1.0 / (L - 1), jnp.float32)
    inv1 = jnp.full((LANES,), 1.0 / L, jnp.float32)

    for g in range(groups):
        b = g & 1
        in_copies[b].wait()
        if g + 1 < groups:
            in_copies[1 - b] = start_in(g + 1, 1 - b)
        if out_copies[b] is not None:
            out_copies[b].wait()

        def row_body(r, _):
            rowbase = b * seq_blk + r * L
            rstage = b * stage_blk + r * NFEAT

            def windows(p):
                a = seq_v[pl.ds(p, LANES)]
                bb = seq_v[pl.ds(p + 1, LANES)]
                cc = seq_v[pl.ds(p + 2, LANES)]
                dd = seq_v[pl.ds(p + 3, LANES)]
                return a, bb, cc, dd

            rot1 = (((iota + 1) & (LANES - 1)))[:, None]
            first_b = seq_v[pl.ds(rowbase + 1, LANES)]

            @plsc.parallel_loop(0, n_chunks - 1, unroll=4, carry=first_b)
            def chunk_body(ch, bcur):
                p = rowbase + ch * LANES
                a = seq_v[pl.ds(p, LANES)]
                dd = seq_v[pl.ds(p + 3, LANES)]
                bnext = seq_v[pl.ds(p + LANES + 1, LANES)]
                cc = jnp.where(iota < LANES - 1, vrot(bcur, rot1),
                               vrot(bnext, rot1))
                code = (((a << 2) + bcur) << 2) + cc
                code = (code << 2) + dd
                plsc.addupdate_scatter(hist4_v, [code], ones_f)
                return bnext

            a, bb, cc, dd = windows(rowbase + (n_chunks - 1) * LANES)
            t1 = (a << 2) + bb
            code = (((t1 << 2) + cc) << 2) + dd
            plsc.addupdate_scatter(hist4_v, [code], ones_f,
                                   mask=iota < LANES - 3)
            v3 = (t1 << 2) + cc + H3
            v2 = t1 + H2
            v1 = a + H1
            corr = jnp.where(iota == LANES - 3, v3,
                             jnp.where(iota == LANES - 2, v2, v1))
            plsc.addupdate_scatter(hist123_v, [corr], ones_f,
                                   mask=iota >= LANES - 3)

            for i in range(16):
                v = hist4_v[pl.ds(i * LANES, LANES)]
                plsc.addupdate_scatter(hist123_v, [quad + (H3 + 4 * i)], v)
                stage_v[pl.ds(rstage + O4 + i * LANES, LANES)] = v * inv4
                hist4_v[pl.ds(i * LANES, LANES)] = zeros_f
            for i in range(4):
                v = hist123_v[pl.ds(H3 + i * LANES, LANES)]
                plsc.addupdate_scatter(hist123_v, [quad + (H2 + 4 * i)], v)
                stage_v[pl.ds(rstage + O3 + i * LANES, LANES)] = v * inv3
                hist123_v[pl.ds(H3 + i * LANES, LANES)] = zeros_f
            v = hist123_v[pl.ds(H2, LANES)]
            plsc.addupdate_scatter(hist123_v, [quad + H1], v)
            stage_v[pl.ds(rstage + O2, LANES)] = v * inv2
            hist123_v[pl.ds(H2, LANES)] = zeros_f
            v = hist123_v[pl.ds(H1 - 12, LANES)]
            plsc.store_scatter(stage_v, [(iota - 12) + rstage], v * inv1,
                               mask=iota >= 12)
            hist123_v[pl.ds(H1 - 12, LANES)] = zeros_f
            return 0

        lax.fori_loop(0, LANES, row_body, 0)

        out_copies[b] = pltpu.async_copy(
            stage_v.at[pl.ds(b * stage_blk, stage_blk)],
            out_hbm.at[pl.ds((row0 + g * LANES) * NFEAT, stage_blk)],
            sems_out[b])

    for cp in out_copies:
        if cp is not None:
            cp.wait()


def kernel(sequences):
    B, L = sequences.shape
    groups = B // (32 * LANES)
    mesh = plsc.VectorSubcoreMesh(core_axis_name="c", subcore_axis_name="s")
    run = pl.kernel(
        functools.partial(_encoder_body, L, groups),
        out_type=jax.ShapeDtypeStruct((B * NFEAT,), jnp.float32),
        mesh=mesh,
        scratch_types=[
            pltpu.VMEM((2 * LANES * L,), jnp.int32),
            pltpu.VMEM((256,), jnp.float32),
            pltpu.VMEM((96,), jnp.float32),
            pltpu.VMEM((2 * LANES * NFEAT,), jnp.float32),
            pltpu.SemaphoreType.DMA,
            pltpu.SemaphoreType.DMA,
            pltpu.SemaphoreType.DMA,
            pltpu.SemaphoreType.DMA,
        ],
        compiler_params=pltpu.CompilerParams(
            needs_layout_passes=False, disable_bounds_checks=True),
    )
    out = run(sequences.reshape(B * L))
    return out.reshape(B, NFEAT)
